# Initial kernel scaffold; baseline (speedup 1.0000x reference)
import jax
import jax.numpy as jnp
from jax.experimental import pallas as pl


def kernel(x, params, era_latlons, h_latlons, e2h_edge_attr, h2e_edge_attr, e2h_edge_index, h2e_edge_index):
    raise NotImplementedError("write your pallas kernel here")


# speedup vs baseline: 1.1639x; 1.1639x over previous
import functools

import jax
import jax.numpy as jnp
from jax import lax
from jax.experimental import pallas as pl
from jax.experimental.pallas import tpu as pltpu

N_ERA = 50000
N_H = 10000
E = 160000
IN_CH = 128
HID = 128

_INTERPRET = False


def _ln(x, g, b):
    mu = jnp.mean(x, axis=-1, keepdims=True)
    var = jnp.mean((x - mu) ** 2, axis=-1, keepdims=True)
    return (x - mu) * jax.lax.rsqrt(var + 1e-5) * g + b


def _silu(x):
    return x * jax.nn.sigmoid(x)


def _dot(a, b):
    return jnp.dot(a, b, preferred_element_type=jnp.float32)


def _edge_embed_body(attr, w1, b1, w2, b2, g, bln, e_out):
    h = _silu(_dot(attr[...], w1[...]) + b1[...])
    e_out[...] = _ln(_dot(h, w2[...]) + b2[...], g[...], bln[...])


def _edge_embed(attr, p, rb=2000):
    n = attr.shape[0]
    grid = (n // rb,)
    full = lambda shp: pl.BlockSpec(shp, lambda i: (0, 0))
    return pl.pallas_call(
        _edge_embed_body,
        grid=grid,
        in_specs=[
            pl.BlockSpec((rb, 4), lambda i: (i, 0)),
            full((4, HID)), full((1, HID)), full((HID, HID)), full((1, HID)),
            full((1, HID)), full((1, HID)),
        ],
        out_specs=pl.BlockSpec((rb, HID), lambda i: (i, 0)),
        out_shape=jax.ShapeDtypeStruct((n, HID), jnp.float32),
        interpret=_INTERPRET,
    )(attr, p['w1'], p['b1'].reshape(1, -1), p['w2'], p['b2'].reshape(1, -1),
      p['g'].reshape(1, -1), p['bln'].reshape(1, -1))


def _src_embed_body(x, ll, w1x, w1l, b1, w2, b2, g, bln, a_w, bdec_w,
                    xs_out, ps_out, pd_out):
    h = _silu(_dot(x[...], w1x[...]) + _dot(ll[...], w1l[...]) + b1[...])
    xs = _ln(_dot(h, w2[...]) + b2[...], g[...], bln[...])
    xs_out[...] = xs
    ps_out[...] = _dot(xs, a_w[...])
    pd_out[...] = _dot(xs, bdec_w[...])


def _src_embed(x, ll, p, a_w, bdec_w, rb=2000):
    n = x.shape[0]
    grid = (n // rb,)
    full = lambda shp: pl.BlockSpec(shp, lambda i: (0, 0))
    return pl.pallas_call(
        _src_embed_body,
        grid=grid,
        in_specs=[
            pl.BlockSpec((rb, IN_CH), lambda i: (i, 0)),
            pl.BlockSpec((rb, 4), lambda i: (i, 0)),
            full((IN_CH, HID)), full((4, HID)), full((1, HID)),
            full((HID, HID)), full((1, HID)), full((1, HID)), full((1, HID)),
            full((HID, HID)), full((HID, HID)),
        ],
        out_specs=[pl.BlockSpec((rb, HID), lambda i: (i, 0))] * 3,
        out_shape=[jax.ShapeDtypeStruct((n, HID), jnp.float32)] * 3,
        interpret=_INTERPRET,
    )(x, ll, p['w1'][:IN_CH], p['w1'][IN_CH:], p['b1'].reshape(1, -1),
      p['w2'], p['b2'].reshape(1, -1), p['g'].reshape(1, -1),
      p['bln'].reshape(1, -1), a_w, bdec_w)


def _dst_embed_body(ll, w1, b1, w2, b2, g, bln, benc_w, xd_out, pd_out):
    h = _silu(_dot(ll[...], w1[...]) + b1[...])
    xd = _ln(_dot(h, w2[...]) + b2[...], g[...], bln[...])
    xd_out[...] = xd
    pd_out[...] = _dot(xd, benc_w[...])


def _dst_embed(ll, p, benc_w, rb=2000):
    n = ll.shape[0]
    grid = (n // rb,)
    full = lambda shp: pl.BlockSpec(shp, lambda i: (0, 0))
    return pl.pallas_call(
        _dst_embed_body,
        grid=grid,
        in_specs=[
            pl.BlockSpec((rb, 4), lambda i: (i, 0)),
            full((4, HID)), full((1, HID)), full((HID, HID)), full((1, HID)),
            full((1, HID)), full((1, HID)), full((HID, HID)),
        ],
        out_specs=[pl.BlockSpec((rb, HID), lambda i: (i, 0))] * 2,
        out_shape=[jax.ShapeDtypeStruct((n, HID), jnp.float32)] * 2,
        interpret=_INTERPRET,
    )(ll, p['w1'], p['b1'].reshape(1, -1), p['w2'], p['b2'].reshape(1, -1),
      p['g'].reshape(1, -1), p['bln'].reshape(1, -1), benc_w)


def _edge_msg_body(sgd, e, c_w, b1, w2, b2, g, bln, m_out):
    h = _silu(sgd[...] + _dot(e[...], c_w[...]) + b1[...])
    m_out[...] = _ln(_dot(h, w2[...]) + b2[...], g[...], bln[...]) + e[...]


def _edge_msg(sgd, e, p, rb=2000):
    n = sgd.shape[0]
    grid = (n // rb,)
    full = lambda shp: pl.BlockSpec(shp, lambda i: (0, 0))
    return pl.pallas_call(
        _edge_msg_body,
        grid=grid,
        in_specs=[
            pl.BlockSpec((rb, HID), lambda i: (i, 0)),
            pl.BlockSpec((rb, HID), lambda i: (i, 0)),
            full((HID, HID)), full((1, HID)), full((HID, HID)), full((1, HID)),
            full((1, HID)), full((1, HID)),
        ],
        out_specs=pl.BlockSpec((rb, HID), lambda i: (i, 0)),
        out_shape=jax.ShapeDtypeStruct((n, HID), jnp.float32),
        interpret=_INTERPRET,
    )(sgd, e, p['w1'][2 * HID:], p['b1'].reshape(1, -1), p['w2'],
      p['b2'].reshape(1, -1), p['g'].reshape(1, -1), p['bln'].reshape(1, -1))


def _node_update_body(project, xd, agg, v1a, v1b, b1, w2, b2, g, bln, pw, pb,
                      out0, out1=None):
    h = _silu(_dot(xd[...], v1a[...]) + _dot(agg[...], v1b[...]) + b1[...])
    xn = xd[...] + _ln(_dot(h, w2[...]) + b2[...], g[...], bln[...])
    if project:
        out0[...] = _dot(xn, pw[...]) + pb[...]
    else:
        out0[...] = xn
        out1[...] = _dot(xn, pw[...]) + pb[...]


def _node_update(xd, agg, p, pw, pb, project, rb=2000):
    n = xd.shape[0]
    grid = (n // rb,)
    full = lambda shp: pl.BlockSpec(shp, lambda i: (0, 0))
    pout = pw.shape[1]
    if project:
        out_specs = pl.BlockSpec((rb, pout), lambda i: (i, 0))
        out_shape = jax.ShapeDtypeStruct((n, pout), jnp.float32)
    else:
        out_specs = [pl.BlockSpec((rb, HID), lambda i: (i, 0)),
                     pl.BlockSpec((rb, pout), lambda i: (i, 0))]
        out_shape = [jax.ShapeDtypeStruct((n, HID), jnp.float32),
                     jax.ShapeDtypeStruct((n, pout), jnp.float32)]
    return pl.pallas_call(
        functools.partial(_node_update_body, project),
        grid=grid,
        in_specs=[
            pl.BlockSpec((rb, HID), lambda i: (i, 0)),
            pl.BlockSpec((rb, HID), lambda i: (i, 0)),
            full((HID, HID)), full((HID, HID)), full((1, HID)),
            full((HID, HID)), full((1, HID)), full((1, HID)), full((1, HID)),
            full((HID, pout)), full((1, pout)),
        ],
        out_specs=out_specs,
        out_shape=out_shape,
        interpret=_INTERPRET,
    )(xd, agg, p['w1'][:HID], p['w1'][HID:], p['b1'].reshape(1, -1),
      p['w2'], p['b2'].reshape(1, -1), p['g'].reshape(1, -1),
      p['bln'].reshape(1, -1), pw, pb.reshape(1, -1))


def _gather_add(ps, pd, src_idx, dst_idx):
    return ps[src_idx] + pd[dst_idx]


def _seg_sum(m, dst_idx, n_seg):
    return jax.ops.segment_sum(m, dst_idx, num_segments=n_seg)


def kernel(x, params, era_latlons, h_latlons, e2h_edge_attr, h2e_edge_attr,
           e2h_edge_index, h2e_edge_index):
    enc, dec = params['enc'], params['dec']
    bs = x.shape[0]
    x_flat = x.reshape(bs * N_ERA, IN_CH)

    a_enc = enc['blk0_edge']['w1'][:HID]
    b_enc = enc['blk0_edge']['w1'][HID:2 * HID]
    a_dec = dec['blk0_edge']['w1'][:HID]
    b_dec = dec['blk0_edge']['w1'][HID:2 * HID]

    e1 = _edge_embed(e2h_edge_attr, enc['emb_edges'])
    xs, ps1, pd2 = _src_embed(x_flat, era_latlons, enc['emb_src'],
                              a_enc, b_dec)
    xd, pd1 = _dst_embed(h_latlons, enc['emb_dst'], b_enc)

    sgd1 = _gather_add(ps1, pd1, e2h_edge_index[0], e2h_edge_index[1])
    m1 = _edge_msg(sgd1, e1, enc['blk0_edge'])
    agg1 = _seg_sum(m1, e2h_edge_index[1], N_H)
    xlat, ps2 = _node_update(xd, agg1, enc['blk0_node'], a_dec,
                             jnp.zeros((HID,), jnp.float32), project=False)

    e2 = _edge_embed(h2e_edge_attr, dec['emb_edges'])
    sgd2 = _gather_add(ps2, pd2, h2e_edge_index[0], h2e_edge_index[1])
    m2 = _edge_msg(sgd2, e2, dec['blk0_edge'])
    agg2 = _seg_sum(m2, h2e_edge_index[1], N_ERA)
    out = _node_update(xs, agg2, dec['blk0_node'], dec['out_w'],
                       dec['out_b'], project=True)
    return out.reshape(bs, N_ERA, IN_CH)


# speedup vs baseline: 1.7482x; 1.5021x over previous
import functools

import jax
import jax.numpy as jnp
from jax import lax
from jax.experimental import pallas as pl
from jax.experimental.pallas import tpu as pltpu
from jax.experimental.pallas import tpu_sc as plsc

N_ERA = 50000
N_H = 10000
E = 160000
IN_CH = 128
HID = 128

_INTERPRET = False


def _ln(x, g, b):
    mu = jnp.mean(x, axis=-1, keepdims=True)
    var = jnp.mean((x - mu) ** 2, axis=-1, keepdims=True)
    return (x - mu) * jax.lax.rsqrt(var + 1e-5) * g + b


def _silu(x):
    return x * jax.nn.sigmoid(x)


def _dot(a, b):
    return jnp.dot(a, b, preferred_element_type=jnp.float32)


def _edge_embed_body(attr, w1, b1, w2, b2, g, bln, e_out):
    h = _silu(_dot(attr[...], w1[...]) + b1[...])
    e_out[...] = _ln(_dot(h, w2[...]) + b2[...], g[...], bln[...])


def _edge_embed(attr, p, rb=2000):
    n = attr.shape[0]
    grid = (n // rb,)
    full = lambda shp: pl.BlockSpec(shp, lambda i: (0, 0))
    return pl.pallas_call(
        _edge_embed_body,
        grid=grid,
        in_specs=[
            pl.BlockSpec((rb, 4), lambda i: (i, 0)),
            full((4, HID)), full((1, HID)), full((HID, HID)), full((1, HID)),
            full((1, HID)), full((1, HID)),
        ],
        out_specs=pl.BlockSpec((rb, HID), lambda i: (i, 0)),
        out_shape=jax.ShapeDtypeStruct((n, HID), jnp.float32),
        interpret=_INTERPRET,
    )(attr, p['w1'], p['b1'].reshape(1, -1), p['w2'], p['b2'].reshape(1, -1),
      p['g'].reshape(1, -1), p['bln'].reshape(1, -1))


def _src_embed_body(x, ll, w1x, w1l, b1, w2, b2, g, bln, a_w, bdec_w,
                    xs_out, ps_out, pd_out):
    h = _silu(_dot(x[...], w1x[...]) + _dot(ll[...], w1l[...]) + b1[...])
    xs = _ln(_dot(h, w2[...]) + b2[...], g[...], bln[...])
    xs_out[...] = xs
    ps_out[...] = _dot(xs, a_w[...])
    pd_out[...] = _dot(xs, bdec_w[...])


def _src_embed(x, ll, p, a_w, bdec_w, rb=2000):
    n = x.shape[0]
    grid = (n // rb,)
    full = lambda shp: pl.BlockSpec(shp, lambda i: (0, 0))
    return pl.pallas_call(
        _src_embed_body,
        grid=grid,
        in_specs=[
            pl.BlockSpec((rb, IN_CH), lambda i: (i, 0)),
            pl.BlockSpec((rb, 4), lambda i: (i, 0)),
            full((IN_CH, HID)), full((4, HID)), full((1, HID)),
            full((HID, HID)), full((1, HID)), full((1, HID)), full((1, HID)),
            full((HID, HID)), full((HID, HID)),
        ],
        out_specs=[pl.BlockSpec((rb, HID), lambda i: (i, 0))] * 3,
        out_shape=[jax.ShapeDtypeStruct((n, HID), jnp.float32)] * 3,
        interpret=_INTERPRET,
    )(x, ll, p['w1'][:IN_CH], p['w1'][IN_CH:], p['b1'].reshape(1, -1),
      p['w2'], p['b2'].reshape(1, -1), p['g'].reshape(1, -1),
      p['bln'].reshape(1, -1), a_w, bdec_w)


def _dst_embed_body(ll, w1, b1, w2, b2, g, bln, benc_w, xd_out, pd_out):
    h = _silu(_dot(ll[...], w1[...]) + b1[...])
    xd = _ln(_dot(h, w2[...]) + b2[...], g[...], bln[...])
    xd_out[...] = xd
    pd_out[...] = _dot(xd, benc_w[...])


def _dst_embed(ll, p, benc_w, rb=2000):
    n = ll.shape[0]
    grid = (n // rb,)
    full = lambda shp: pl.BlockSpec(shp, lambda i: (0, 0))
    return pl.pallas_call(
        _dst_embed_body,
        grid=grid,
        in_specs=[
            pl.BlockSpec((rb, 4), lambda i: (i, 0)),
            full((4, HID)), full((1, HID)), full((HID, HID)), full((1, HID)),
            full((1, HID)), full((1, HID)), full((HID, HID)),
        ],
        out_specs=[pl.BlockSpec((rb, HID), lambda i: (i, 0))] * 2,
        out_shape=[jax.ShapeDtypeStruct((n, HID), jnp.float32)] * 2,
        interpret=_INTERPRET,
    )(ll, p['w1'], p['b1'].reshape(1, -1), p['w2'], p['b2'].reshape(1, -1),
      p['g'].reshape(1, -1), p['bln'].reshape(1, -1), benc_w)


def _edge_msg_body(sgd, e, c_w, b1, w2, b2, g, bln, m_out):
    h = _silu(sgd[...] + _dot(e[...], c_w[...]) + b1[...])
    m_out[...] = _ln(_dot(h, w2[...]) + b2[...], g[...], bln[...]) + e[...]


def _edge_msg(sgd, e, p, rb=2000):
    n = sgd.shape[0]
    grid = (n // rb,)
    full = lambda shp: pl.BlockSpec(shp, lambda i: (0, 0))
    return pl.pallas_call(
        _edge_msg_body,
        grid=grid,
        in_specs=[
            pl.BlockSpec((rb, HID), lambda i: (i, 0)),
            pl.BlockSpec((rb, HID), lambda i: (i, 0)),
            full((HID, HID)), full((1, HID)), full((HID, HID)), full((1, HID)),
            full((1, HID)), full((1, HID)),
        ],
        out_specs=pl.BlockSpec((rb, HID), lambda i: (i, 0)),
        out_shape=jax.ShapeDtypeStruct((n, HID), jnp.float32),
        interpret=_INTERPRET,
    )(sgd, e, p['w1'][2 * HID:], p['b1'].reshape(1, -1), p['w2'],
      p['b2'].reshape(1, -1), p['g'].reshape(1, -1), p['bln'].reshape(1, -1))


def _node_update_body(project, xd, agg, v1a, v1b, b1, w2, b2, g, bln, pw, pb,
                      out0, out1=None):
    h = _silu(_dot(xd[...], v1a[...]) + _dot(agg[...], v1b[...]) + b1[...])
    xn = xd[...] + _ln(_dot(h, w2[...]) + b2[...], g[...], bln[...])
    if project:
        out0[...] = _dot(xn, pw[...]) + pb[...]
    else:
        out0[...] = xn
        out1[...] = _dot(xn, pw[...]) + pb[...]


def _node_update(xd, agg, p, pw, pb, project, rb=2000):
    n = xd.shape[0]
    grid = (n // rb,)
    full = lambda shp: pl.BlockSpec(shp, lambda i: (0, 0))
    pout = pw.shape[1]
    if project:
        out_specs = pl.BlockSpec((rb, pout), lambda i: (i, 0))
        out_shape = jax.ShapeDtypeStruct((n, pout), jnp.float32)
    else:
        out_specs = [pl.BlockSpec((rb, HID), lambda i: (i, 0)),
                     pl.BlockSpec((rb, pout), lambda i: (i, 0))]
        out_shape = [jax.ShapeDtypeStruct((n, HID), jnp.float32),
                     jax.ShapeDtypeStruct((n, pout), jnp.float32)]
    return pl.pallas_call(
        functools.partial(_node_update_body, project),
        grid=grid,
        in_specs=[
            pl.BlockSpec((rb, HID), lambda i: (i, 0)),
            pl.BlockSpec((rb, HID), lambda i: (i, 0)),
            full((HID, HID)), full((HID, HID)), full((1, HID)),
            full((HID, HID)), full((1, HID)), full((1, HID)), full((1, HID)),
            full((HID, pout)), full((1, pout)),
        ],
        out_specs=out_specs,
        out_shape=out_shape,
        interpret=_INTERPRET,
    )(xd, agg, p['w1'][:HID], p['w1'][HID:], p['b1'].reshape(1, -1),
      p['w2'], p['b2'].reshape(1, -1), p['g'].reshape(1, -1),
      p['bln'].reshape(1, -1), pw, pb.reshape(1, -1))


_NC = 2
_NS = 16
_NW = _NC * _NS


def _gather_add(ps, pd, src_idx, dst_idx):
    n = src_idx.shape[0]
    ch = n // _NW
    K = 200
    nch = ch // K
    assert ch * _NW == n and nch * K == ch and K % 8 == 0

    mesh = plsc.VectorSubcoreMesh(core_axis_name="c", subcore_axis_name="s")

    @functools.partial(
        pl.kernel, mesh=mesh,
        out_type=jax.ShapeDtypeStruct((n, HID), jnp.float32),
        scratch_types=[
            pltpu.VMEM((K,), jnp.int32),
            pltpu.VMEM((K,), jnp.int32),
            pltpu.VMEM((K, HID), jnp.float32),
            pltpu.VMEM((K, HID), jnp.float32),
            pltpu.SemaphoreType.DMA,
            pltpu.SemaphoreType.DMA,
        ],
    )
    def k(ps_hbm, pd_hbm, si_hbm, di_hbm, out_hbm, si_v, di_v, ra, rb, sa, sb):
        wid = lax.axis_index("s") * _NC + lax.axis_index("c")
        base0 = wid * ch

        def chunk(i, carry):
            base = base0 + i * K
            pltpu.sync_copy(si_hbm.at[pl.ds(base, K)], si_v)
            pltpu.sync_copy(di_hbm.at[pl.ds(base, K)], di_v)
            ca = pltpu.async_copy(ps_hbm.at[si_v], ra, sa)
            cb = pltpu.async_copy(pd_hbm.at[di_v], rb, sb)
            ca.wait()
            cb.wait()

            def row(r, c2):
                def col(j, c3):
                    sl = pl.ds(j * 16, 16)
                    rb[r, sl] = ra[r, sl] + rb[r, sl]
                    return c3
                return lax.fori_loop(0, HID // 16, col, c2)
            lax.fori_loop(0, K, row, 0)
            pltpu.sync_copy(rb, out_hbm.at[pl.ds(base, K)])
            return carry
        lax.fori_loop(0, nch, chunk, 0)

    return k(ps, pd, src_idx, dst_idx)


def _seg_sum(m, dst_idx, n_seg):
    return jax.ops.segment_sum(m, dst_idx, num_segments=n_seg)


def kernel(x, params, era_latlons, h_latlons, e2h_edge_attr, h2e_edge_attr,
           e2h_edge_index, h2e_edge_index):
    enc, dec = params['enc'], params['dec']
    bs = x.shape[0]
    x_flat = x.reshape(bs * N_ERA, IN_CH)

    a_enc = enc['blk0_edge']['w1'][:HID]
    b_enc = enc['blk0_edge']['w1'][HID:2 * HID]
    a_dec = dec['blk0_edge']['w1'][:HID]
    b_dec = dec['blk0_edge']['w1'][HID:2 * HID]

    e1 = _edge_embed(e2h_edge_attr, enc['emb_edges'])
    xs, ps1, pd2 = _src_embed(x_flat, era_latlons, enc['emb_src'],
                              a_enc, b_dec)
    xd, pd1 = _dst_embed(h_latlons, enc['emb_dst'], b_enc)

    sgd1 = _gather_add(ps1, pd1, e2h_edge_index[0], e2h_edge_index[1])
    m1 = _edge_msg(sgd1, e1, enc['blk0_edge'])
    agg1 = _seg_sum(m1, e2h_edge_index[1], N_H)
    xlat, ps2 = _node_update(xd, agg1, enc['blk0_node'], a_dec,
                             jnp.zeros((HID,), jnp.float32), project=False)

    e2 = _edge_embed(h2e_edge_attr, dec['emb_edges'])
    sgd2 = _gather_add(ps2, pd2, h2e_edge_index[0], h2e_edge_index[1])
    m2 = _edge_msg(sgd2, e2, dec['blk0_edge'])
    agg2 = _seg_sum(m2, h2e_edge_index[1], N_ERA)
    out = _node_update(xs, agg2, dec['blk0_node'], dec['out_w'],
                       dec['out_b'], project=True)
    return out.reshape(bs, N_ERA, IN_CH)
